# Initial kernel scaffold; baseline (speedup 1.0000x reference)
import jax
import jax.numpy as jnp
from jax.experimental import pallas as pl


def kernel(x, edge_index, cluster, W1_0, b1_0, g_0, be_0, W2_0, b2_0, W1_1, b1_1, g_1, be_1, W2_1, b2_1, W1_2, b1_2, g_2, be_2, W2_2, b2_2):
    raise NotImplementedError("write your pallas kernel here")


# speedup vs baseline: 1.5584x; 1.5584x over previous
import functools

import jax
import jax.numpy as jnp
from jax import lax
from jax.experimental import pallas as pl
from jax.experimental.pallas import tpu as pltpu
from jax.experimental.pallas import tpu_sc as plsc

_N = 10000
_E = 320000
_NCLU = 1000
_H = 64

_NC = 2
_NS = 16
_NW = _NC * _NS

_CH = 128
_G = 64
_NGE = 5
_NPW = _G * _NGE
_NPAD = _NW * _NPW

_GC = 32
_CPAD = _NW * _GC

_SENT = 1 << 30


def _mlp_body(x1_ref, x2_ref, w1a_ref, w1b_ref, b1_ref, g_ref, be_ref,
              w2_ref, b2_ref, out_ref):
    dot = functools.partial(jnp.dot, preferred_element_type=jnp.float32,
                            precision=lax.Precision.HIGHEST)
    h = dot(x1_ref[...], w1a_ref[...]) + dot(x2_ref[...], w1b_ref[...])
    h = h + b1_ref[...]
    mu = jnp.mean(h, axis=-1, keepdims=True)
    var = jnp.mean((h - mu) * (h - mu), axis=-1, keepdims=True)
    h = (h - mu) * lax.rsqrt(var + 1e-5) * g_ref[...] + be_ref[...]
    h = jnp.maximum(h, 0.0)
    out_ref[...] = dot(h, w2_ref[...]) + b2_ref[...]


def _mlp(x1, x2, w1a, w1b, b1, g, be, w2, b2):
    c_in = x1.shape[1]
    c_out = w2.shape[1]
    rows = x1.shape[0]
    blk = 512
    grid = rows // blk
    return pl.pallas_call(
        _mlp_body,
        grid=(grid,),
        in_specs=[
            pl.BlockSpec((blk, c_in), lambda i: (i, 0)),
            pl.BlockSpec((blk, c_in), lambda i: (i, 0)),
            pl.BlockSpec((c_in, _H), lambda i: (0, 0)),
            pl.BlockSpec((c_in, _H), lambda i: (0, 0)),
            pl.BlockSpec((1, _H), lambda i: (0, 0)),
            pl.BlockSpec((1, _H), lambda i: (0, 0)),
            pl.BlockSpec((1, _H), lambda i: (0, 0)),
            pl.BlockSpec((_H, c_out), lambda i: (0, 0)),
            pl.BlockSpec((1, c_out), lambda i: (0, 0)),
        ],
        out_specs=pl.BlockSpec((blk, c_out), lambda i: (i, 0)),
        out_shape=jax.ShapeDtypeStruct((rows, c_out), jnp.float32),
    )(x1, x2, w1a, w1b, b1.reshape(1, -1), g.reshape(1, -1),
      be.reshape(1, -1), w2, b2.reshape(1, -1))


def _make_segmax(c, group, ngroups, use_gather):
    npw = group * ngroups
    out_rows = _NW * npw
    nstr = c // 16
    mesh = plsc.VectorSubcoreMesh(core_axis_name="c", subcore_axis_name="s",
                                  num_cores=_NC, num_subcores=_NS)

    @functools.partial(
        pl.kernel,
        out_type=jax.ShapeDtypeStruct((out_rows, c), jnp.float32),
        mesh=mesh,
        scratch_types=[
            pltpu.VMEM((_CH + 16,), jnp.int32),
            pltpu.VMEM((_CH,), jnp.int32),
            pltpu.VMEM((_CH, c), jnp.float32),
            pltpu.VMEM((group, c), jnp.float32),
            pltpu.VMEM((16,), jnp.int32),
            pltpu.SemaphoreType.DMA,
        ],
    )
    def segmax(h_hbm, segp_hbm, srcp_hbm, gb_hbm, out_hbm,
               seg_v, src_v, rows_v, stage, gb_v, sem):
        wid = lax.axis_index("s") * _NC + lax.axis_index("c")
        pltpu.sync_copy(gb_hbm.at[wid], gb_v)
        gbrow = gb_v[...]

        for gi in range(ngroups):
            n_lo = wid * npw + gi * group
            es = gbrow[gi]
            ee = gbrow[gi + 1]

            def init_body(r, _):
                for s in range(nstr):
                    stage[r, pl.ds(s * 16, 16)] = jnp.full((16,), -jnp.inf,
                                                           jnp.float32)
                return 0
            lax.fori_loop(0, group, init_body, 0)

            e0 = es & jnp.int32(-8)
            nch = (ee - e0 + _CH - 1) >> 7

            def chunk_body(i, _):
                base = pl.multiple_of(e0 + i * _CH, 8)
                pltpu.sync_copy(segp_hbm.at[pl.ds(base, _CH)],
                                seg_v.at[pl.ds(0, _CH)])
                if use_gather:
                    pltpu.sync_copy(srcp_hbm.at[pl.ds(base, _CH)], src_v)
                    pltpu.async_copy(h_hbm.at[src_v], rows_v, sem).wait()
                else:
                    pltpu.sync_copy(h_hbm.at[pl.ds(base, _CH)], rows_v)

                def edge_body(k, _):
                    seg = seg_v[pl.ds(k, 16)][0]
                    loc = seg - n_lo
                    ok = (loc >= 0) & (loc < group)

                    @pl.when(ok)
                    def _():
                        for s in range(nstr):
                            sl = pl.ds(s * 16, 16)
                            stage[loc, sl] = jnp.maximum(stage[loc, sl],
                                                         rows_v[k, sl])
                    return 0
                lax.fori_loop(0, _CH, edge_body, 0)
                return 0
            lax.fori_loop(0, nch, chunk_body, 0)

            def fix_body(r, _):
                for s in range(nstr):
                    sl = pl.ds(s * 16, 16)
                    v = stage[r, sl]
                    stage[r, sl] = jnp.where(v == -jnp.inf, 0.0, v)
                return 0
            lax.fori_loop(0, group, fix_body, 0)
            pltpu.sync_copy(stage, out_hbm.at[pl.ds(n_lo, group)])

    return segmax


_segmax_e = {c: _make_segmax(c, _G, _NGE, True) for c in (128, 256, 512)}
_segmax_clu = _make_segmax(512, _GC, 1, False)


def kernel(x, edge_index, cluster,
           W1_0, b1_0, g_0, be_0, W2_0, b2_0,
           W1_1, b1_1, g_1, be_1, W2_1, b2_1,
           W1_2, b1_2, g_2, be_2, W2_2, b2_2):
    src = edge_index[0]
    dst = edge_index[1]
    perm = jnp.argsort(dst)
    dst_s = dst[perm]
    src_s = src[perm]

    pad = jnp.full((2 * _CH,), _SENT, jnp.int32)
    segp = jnp.concatenate([dst_s, pad])
    srcp = jnp.concatenate([src_s, jnp.zeros((2 * _CH,), jnp.int32)])
    clup = jnp.concatenate(
        [cluster, jnp.full((_NPAD - _N + 2 * _CH,), _SENT, jnp.int32)])

    gb_nodes = jnp.minimum(
        (jnp.arange(_NW)[:, None] * _NPW
         + jnp.arange(16)[None, :] * _G), _N).astype(jnp.int32)
    gb_e = jnp.searchsorted(dst_s, gb_nodes.ravel(),
                            side="left").astype(jnp.int32).reshape(_NW, 16)
    gc_nodes = jnp.minimum(
        (jnp.arange(_NW)[:, None] * _GC
         + jnp.arange(16)[None, :] * _GC), _NCLU).astype(jnp.int32)
    gb_c = jnp.searchsorted(cluster, gc_nodes.ravel(),
                            side="left").astype(jnp.int32).reshape(_NW, 16)

    xp = jnp.pad(x, ((0, _NPAD - _N), (0, 0)))
    zW = jnp.zeros_like(W1_0)

    h1 = _mlp(xp, xp, W1_0, zW, b1_0, g_0, be_0, W2_0, b2_0)
    a1 = _segmax_e[128](h1, segp, srcp, gb_e)
    h2 = _mlp(h1, a1, W1_1[:128], W1_1[128:], b1_1, g_1, be_1, W2_1, b2_1)
    a2 = _segmax_e[256](h2, segp, srcp, gb_e)
    h3 = _mlp(h2, a2, W1_2[:256], W1_2[256:], b1_2, g_2, be_2, W2_2, b2_2)
    a3 = _segmax_e[512](h3, segp, srcp, gb_e)

    ph = _segmax_clu(h3, clup, clup, gb_c)
    pa = _segmax_clu(a3, clup, clup, gb_c)
    return jnp.concatenate([ph[:_NCLU], pa[:_NCLU]], axis=1)


# speedup vs baseline: 2.8147x; 1.8062x over previous
import functools

import jax
import jax.numpy as jnp
from jax import lax
from jax.experimental import pallas as pl
from jax.experimental.pallas import tpu as pltpu
from jax.experimental.pallas import tpu_sc as plsc

_N = 10000
_E = 320000
_NCLU = 1000
_H = 64

_NC = 2
_NS = 16
_NW = _NC * _NS

_CH = 128
_G = 64
_NGE = 5
_NPW = _G * _NGE
_NPAD = _NW * _NPW

_GC = 32
_CPAD = _NW * _GC

_SENT = 1 << 30


def _mlp_body(x1_ref, x2_ref, w1a_ref, w1b_ref, b1_ref, g_ref, be_ref,
              w2_ref, b2_ref, out_ref):
    dot = functools.partial(jnp.dot, preferred_element_type=jnp.float32,
                            precision=lax.Precision.HIGHEST)
    h = dot(x1_ref[...], w1a_ref[...]) + dot(x2_ref[...], w1b_ref[...])
    h = h + b1_ref[...]
    mu = jnp.mean(h, axis=-1, keepdims=True)
    var = jnp.mean((h - mu) * (h - mu), axis=-1, keepdims=True)
    h = (h - mu) * lax.rsqrt(var + 1e-5) * g_ref[...] + be_ref[...]
    h = jnp.maximum(h, 0.0)
    out_ref[...] = dot(h, w2_ref[...]) + b2_ref[...]


def _mlp(x1, x2, w1a, w1b, b1, g, be, w2, b2):
    c_in = x1.shape[1]
    c_out = w2.shape[1]
    rows = x1.shape[0]
    blk = 512
    grid = rows // blk
    return pl.pallas_call(
        _mlp_body,
        grid=(grid,),
        in_specs=[
            pl.BlockSpec((blk, c_in), lambda i: (i, 0)),
            pl.BlockSpec((blk, c_in), lambda i: (i, 0)),
            pl.BlockSpec((c_in, _H), lambda i: (0, 0)),
            pl.BlockSpec((c_in, _H), lambda i: (0, 0)),
            pl.BlockSpec((1, _H), lambda i: (0, 0)),
            pl.BlockSpec((1, _H), lambda i: (0, 0)),
            pl.BlockSpec((1, _H), lambda i: (0, 0)),
            pl.BlockSpec((_H, c_out), lambda i: (0, 0)),
            pl.BlockSpec((1, c_out), lambda i: (0, 0)),
        ],
        out_specs=pl.BlockSpec((blk, c_out), lambda i: (i, 0)),
        out_shape=jax.ShapeDtypeStruct((rows, c_out), jnp.float32),
    )(x1, x2, w1a, w1b, b1.reshape(1, -1), g.reshape(1, -1),
      be.reshape(1, -1), w2, b2.reshape(1, -1))


def _make_segmax(c, group, ngroups, use_gather):
    npw = group * ngroups
    out_rows = _NW * npw
    nstr = c // 16
    mesh = plsc.VectorSubcoreMesh(core_axis_name="c", subcore_axis_name="s",
                                  num_cores=_NC, num_subcores=_NS)

    @functools.partial(
        pl.kernel,
        out_type=jax.ShapeDtypeStruct((out_rows, c), jnp.float32),
        mesh=mesh,
        compiler_params=pltpu.CompilerParams(needs_layout_passes=False),
        scratch_types=[
            pltpu.VMEM((_CH + 16,), jnp.int32),
            pltpu.VMEM((_CH,), jnp.int32),
            pltpu.VMEM((_CH, c), jnp.float32),
            pltpu.VMEM((group, c), jnp.float32),
            pltpu.VMEM((16,), jnp.int32),
            pltpu.SemaphoreType.DMA,
        ],
    )
    def segmax(h_hbm, segp_hbm, srcp_hbm, gb_hbm, out_hbm,
               seg_v, src_v, rows_v, stage, gb_v, sem):
        wid = lax.axis_index("s") * _NC + lax.axis_index("c")
        pltpu.sync_copy(gb_hbm.at[wid], gb_v)
        gbrow = gb_v[...]
        neg = jnp.full((16,), -jnp.inf, jnp.float32)

        for gi in range(ngroups):
            n_lo = wid * npw + gi * group
            es = gbrow[gi]
            ee = gbrow[gi + 1]

            def init_body(r, _):
                for s in range(nstr):
                    stage[r, pl.ds(s * 16, 16)] = neg
                return 0
            lax.fori_loop(0, group, init_body, 0)

            e0 = es & jnp.int32(-8)
            nch = (ee - e0 + _CH) >> 7

            def chunk_body(i, carry):
                base = pl.multiple_of(e0 + i * _CH, 8)
                pltpu.sync_copy(segp_hbm.at[pl.ds(base, _CH)],
                                seg_v.at[pl.ds(0, _CH)])
                if use_gather:
                    pltpu.sync_copy(srcp_hbm.at[pl.ds(base, _CH)], src_v)
                    pltpu.async_copy(h_hbm.at[src_v], rows_v, sem).wait()
                else:
                    pltpu.sync_copy(h_hbm.at[pl.ds(base, _CH)], rows_v)

                def edge_body(k, carry):
                    cur, acc = carry
                    seg = seg_v[pl.ds(k, 16)][0]
                    isnew = seg != cur
                    curv = jnp.full((16,), cur, jnp.int32)
                    rowv = curv - n_lo
                    maskv = ((jnp.full((16,), seg, jnp.int32) != curv)
                             & (rowv >= 0) & (rowv < group))
                    for s in range(nstr):
                        colv = lax.iota(jnp.int32, 16) + s * 16
                        plsc.store_scatter(stage, [rowv, colv], acc[s],
                                           mask=maskv)
                    acc = tuple(
                        jnp.maximum(jnp.where(isnew, neg, acc[s]),
                                    rows_v[k, pl.ds(s * 16, 16)])
                        for s in range(nstr))
                    return (seg, acc)
                return lax.fori_loop(0, _CH, edge_body, carry)

            init = (jnp.int32(-1), tuple(neg for _ in range(nstr)))
            lax.fori_loop(0, nch, chunk_body, init)

            def fix_body(r, _):
                for s in range(nstr):
                    sl = pl.ds(s * 16, 16)
                    v = stage[r, sl]
                    stage[r, sl] = jnp.where(v == -jnp.inf, 0.0, v)
                return 0
            lax.fori_loop(0, group, fix_body, 0)
            pltpu.sync_copy(stage, out_hbm.at[pl.ds(n_lo, group)])

    return segmax


_segmax_e = {c: _make_segmax(c, _G, _NGE, True) for c in (128, 256, 512)}
_segmax_clu = _make_segmax(512, _GC, 1, False)


def kernel(x, edge_index, cluster,
           W1_0, b1_0, g_0, be_0, W2_0, b2_0,
           W1_1, b1_1, g_1, be_1, W2_1, b2_1,
           W1_2, b1_2, g_2, be_2, W2_2, b2_2):
    src = edge_index[0]
    dst = edge_index[1]
    perm = jnp.argsort(dst)
    dst_s = dst[perm]
    src_s = src[perm]

    pad = jnp.full((2 * _CH,), _SENT, jnp.int32)
    segp = jnp.concatenate([dst_s, pad])
    srcp = jnp.concatenate([src_s, jnp.zeros((2 * _CH,), jnp.int32)])
    clup = jnp.concatenate(
        [cluster, jnp.full((_NPAD - _N + 2 * _CH,), _SENT, jnp.int32)])

    gb_nodes = jnp.minimum(
        (jnp.arange(_NW)[:, None] * _NPW
         + jnp.arange(16)[None, :] * _G), _N).astype(jnp.int32)
    gb_e = jnp.searchsorted(dst_s, gb_nodes.ravel(),
                            side="left").astype(jnp.int32).reshape(_NW, 16)
    gc_nodes = jnp.minimum(
        (jnp.arange(_NW)[:, None] * _GC
         + jnp.arange(16)[None, :] * _GC), _NCLU).astype(jnp.int32)
    gb_c = jnp.searchsorted(cluster, gc_nodes.ravel(),
                            side="left").astype(jnp.int32).reshape(_NW, 16)

    xp = jnp.pad(x, ((0, _NPAD - _N), (0, 0)))
    zW = jnp.zeros_like(W1_0)

    h1 = _mlp(xp, xp, W1_0, zW, b1_0, g_0, be_0, W2_0, b2_0)
    a1 = _segmax_e[128](h1, segp, srcp, gb_e)
    h2 = _mlp(h1, a1, W1_1[:128], W1_1[128:], b1_1, g_1, be_1, W2_1, b2_1)
    a2 = _segmax_e[256](h2, segp, srcp, gb_e)
    h3 = _mlp(h2, a2, W1_2[:256], W1_2[256:], b1_2, g_2, be_2, W2_2, b2_2)
    a3 = _segmax_e[512](h3, segp, srcp, gb_e)

    ph = _segmax_clu(h3, clup, clup, gb_c)
    pa = _segmax_clu(a3, clup, clup, gb_c)
    return jnp.concatenate([ph[:_NCLU], pa[:_NCLU]], axis=1)


# speedup vs baseline: 2.9268x; 1.0398x over previous
import functools

import jax
import jax.numpy as jnp
from jax import lax
from jax.experimental import pallas as pl
from jax.experimental.pallas import tpu as pltpu
from jax.experimental.pallas import tpu_sc as plsc

_N = 10000
_E = 320000
_NCLU = 1000
_H = 64

_NC = 2
_NS = 16
_NW = _NC * _NS

_CH = 128
_G = 64
_NGE = 5
_NPW = _G * _NGE
_NPAD = _NW * _NPW

_GC = 32
_CPAD = _NW * _GC

_SENT = 1 << 30


def _mlp_body(x1_ref, x2_ref, w1a_ref, w1b_ref, b1_ref, g_ref, be_ref,
              w2_ref, b2_ref, out_ref):
    dot = functools.partial(jnp.dot, preferred_element_type=jnp.float32,
                            precision=lax.Precision.HIGHEST)
    h = dot(x1_ref[...], w1a_ref[...]) + dot(x2_ref[...], w1b_ref[...])
    h = h + b1_ref[...]
    mu = jnp.mean(h, axis=-1, keepdims=True)
    var = jnp.mean((h - mu) * (h - mu), axis=-1, keepdims=True)
    h = (h - mu) * lax.rsqrt(var + 1e-5) * g_ref[...] + be_ref[...]
    h = jnp.maximum(h, 0.0)
    out_ref[...] = dot(h, w2_ref[...]) + b2_ref[...]


def _mlp(x1, x2, w1a, w1b, b1, g, be, w2, b2):
    c_in = x1.shape[1]
    c_out = w2.shape[1]
    rows = x1.shape[0]
    blk = 512
    grid = rows // blk
    return pl.pallas_call(
        _mlp_body,
        grid=(grid,),
        in_specs=[
            pl.BlockSpec((blk, c_in), lambda i: (i, 0)),
            pl.BlockSpec((blk, c_in), lambda i: (i, 0)),
            pl.BlockSpec((c_in, _H), lambda i: (0, 0)),
            pl.BlockSpec((c_in, _H), lambda i: (0, 0)),
            pl.BlockSpec((1, _H), lambda i: (0, 0)),
            pl.BlockSpec((1, _H), lambda i: (0, 0)),
            pl.BlockSpec((1, _H), lambda i: (0, 0)),
            pl.BlockSpec((_H, c_out), lambda i: (0, 0)),
            pl.BlockSpec((1, c_out), lambda i: (0, 0)),
        ],
        out_specs=pl.BlockSpec((blk, c_out), lambda i: (i, 0)),
        out_shape=jax.ShapeDtypeStruct((rows, c_out), jnp.float32),
    )(x1, x2, w1a, w1b, b1.reshape(1, -1), g.reshape(1, -1),
      be.reshape(1, -1), w2, b2.reshape(1, -1))


def _make_segmax(c, group, ngroups, use_gather):
    npw = group * ngroups
    out_rows = _NW * npw
    nstr = c // 16
    mesh = plsc.VectorSubcoreMesh(core_axis_name="c", subcore_axis_name="s",
                                  num_cores=_NC, num_subcores=_NS)

    @functools.partial(
        pl.kernel,
        out_type=jax.ShapeDtypeStruct((out_rows, c), jnp.float32),
        mesh=mesh,
        compiler_params=pltpu.CompilerParams(needs_layout_passes=False),
        scratch_types=[
            pltpu.VMEM((_CH + 16,), jnp.int32),
            pltpu.VMEM((_CH,), jnp.int32),
            pltpu.VMEM((_CH, c), jnp.float32),
            pltpu.VMEM((group, c), jnp.float32),
            pltpu.VMEM((16,), jnp.int32),
            pltpu.SemaphoreType.DMA,
        ],
    )
    def segmax(h_hbm, segp_hbm, srcp_hbm, gb_hbm, out_hbm,
               seg_v, src_v, rows_v, stage, gb_v, sem):
        wid = lax.axis_index("s") * _NC + lax.axis_index("c")
        pltpu.sync_copy(gb_hbm.at[wid], gb_v)
        gbrow = gb_v[...]
        neg = jnp.full((16,), -jnp.inf, jnp.float32)
        iota16 = lax.iota(jnp.int32, 16)
        zero16 = jnp.zeros((16,), jnp.int32)
        rot16 = jnp.mod(iota16 + 1, 16)

        def splat0(v):
            return v.at[zero16].get(mode="promise_in_bounds")

        def rot1(v):
            return v.at[rot16].get(mode="promise_in_bounds")

        for gi in range(ngroups):
            n_lo = wid * npw + gi * group
            n_lov = jnp.full((16,), n_lo, jnp.int32)
            groupv = jnp.full((16,), group, jnp.int32)
            es = gbrow[gi]
            ee = gbrow[gi + 1]

            def init_body(r, _):
                for s in range(nstr):
                    stage[r, pl.ds(s * 16, 16)] = neg
                return 0
            lax.fori_loop(0, group, init_body, 0)

            e0 = es & jnp.int32(-8)
            nch = (ee - e0 + _CH) >> 7

            def chunk_body(i, carry):
                base = pl.multiple_of(e0 + i * _CH, 8)
                pltpu.sync_copy(segp_hbm.at[pl.ds(base, _CH)],
                                seg_v.at[pl.ds(0, _CH)])
                if use_gather:
                    pltpu.sync_copy(srcp_hbm.at[pl.ds(base, _CH)], src_v)
                    pltpu.async_copy(h_hbm.at[src_v], rows_v, sem).wait()
                else:
                    pltpu.sync_copy(h_hbm.at[pl.ds(base, _CH)], rows_v)

                def win_body(wi, carry):
                    curv0, acc0 = carry
                    segwin0 = seg_v[pl.ds(wi * 16, 16)]

                    def lane_body(j, carry):
                        curv, acc, segwin = carry
                        segv = splat0(segwin)
                        isnewv = segv != curv
                        rowv = curv - n_lov
                        maskv = isnewv & (rowv >= 0) & (rowv < groupv)
                        for s in range(nstr):
                            plsc.store_scatter(stage, [rowv, iota16 + s * 16],
                                               acc[s], mask=maskv)
                        k = wi * 16 + j
                        acc = tuple(
                            jnp.maximum(jnp.where(isnewv, neg, acc[s]),
                                        rows_v[k, pl.ds(s * 16, 16)])
                            for s in range(nstr))
                        return (segv, acc, rot1(segwin))

                    curv0, acc0, _ = lax.fori_loop(
                        0, 16, lane_body, (curv0, acc0, segwin0))
                    return (curv0, acc0)
                return lax.fori_loop(0, _CH // 16, win_body, carry)

            init = (jnp.full((16,), -1, jnp.int32),
                    tuple(neg for _ in range(nstr)))
            lax.fori_loop(0, nch, chunk_body, init)

            def fix_body(r, _):
                for s in range(nstr):
                    sl = pl.ds(s * 16, 16)
                    v = stage[r, sl]
                    stage[r, sl] = jnp.where(v == -jnp.inf, 0.0, v)
                return 0
            lax.fori_loop(0, group, fix_body, 0)
            pltpu.sync_copy(stage, out_hbm.at[pl.ds(n_lo, group)])

    return segmax


_segmax_e = {c: _make_segmax(c, _G, _NGE, True) for c in (128, 256, 512)}
_segmax_clu = _make_segmax(512, _GC, 1, False)


def kernel(x, edge_index, cluster,
           W1_0, b1_0, g_0, be_0, W2_0, b2_0,
           W1_1, b1_1, g_1, be_1, W2_1, b2_1,
           W1_2, b1_2, g_2, be_2, W2_2, b2_2):
    src = edge_index[0]
    dst = edge_index[1]
    perm = jnp.argsort(dst)
    dst_s = dst[perm]
    src_s = src[perm]

    pad = jnp.full((2 * _CH,), _SENT, jnp.int32)
    segp = jnp.concatenate([dst_s, pad])
    srcp = jnp.concatenate([src_s, jnp.zeros((2 * _CH,), jnp.int32)])
    clup = jnp.concatenate(
        [cluster, jnp.full((_NPAD - _N + 2 * _CH,), _SENT, jnp.int32)])

    gb_nodes = jnp.minimum(
        (jnp.arange(_NW)[:, None] * _NPW
         + jnp.arange(16)[None, :] * _G), _N).astype(jnp.int32)
    gb_e = jnp.searchsorted(dst_s, gb_nodes.ravel(),
                            side="left").astype(jnp.int32).reshape(_NW, 16)
    gc_nodes = jnp.minimum(
        (jnp.arange(_NW)[:, None] * _GC
         + jnp.arange(16)[None, :] * _GC), _NCLU).astype(jnp.int32)
    gb_c = jnp.searchsorted(cluster, gc_nodes.ravel(),
                            side="left").astype(jnp.int32).reshape(_NW, 16)

    xp = jnp.pad(x, ((0, _NPAD - _N), (0, 0)))
    zW = jnp.zeros_like(W1_0)

    h1 = _mlp(xp, xp, W1_0, zW, b1_0, g_0, be_0, W2_0, b2_0)
    a1 = _segmax_e[128](h1, segp, srcp, gb_e)
    h2 = _mlp(h1, a1, W1_1[:128], W1_1[128:], b1_1, g_1, be_1, W2_1, b2_1)
    a2 = _segmax_e[256](h2, segp, srcp, gb_e)
    h3 = _mlp(h2, a2, W1_2[:256], W1_2[256:], b1_2, g_2, be_2, W2_2, b2_2)
    a3 = _segmax_e[512](h3, segp, srcp, gb_e)

    ph = _segmax_clu(h3, clup, clup, gb_c)
    pa = _segmax_clu(a3, clup, clup, gb_c)
    return jnp.concatenate([ph[:_NCLU], pa[:_NCLU]], axis=1)
